# baseline (speedup 1.0000x reference)
import functools

import jax
import jax.numpy as jnp
from jax.experimental import pallas as pl
from jax.experimental.pallas import tpu as pltpu

_EPS = 1e-8


def _fused_block(x_ref, p_ref, a_ref, b_ref, o_ref, *, rank):
    xb = x_ref[:, :]
    p = p_ref[:, :]
    tn = xb.shape[0]
    e = p.shape[0]
    er = a_ref.shape[0]

    xnorm = jnp.sqrt(jnp.sum(xb * xb, axis=1, keepdims=True))
    pnorm = jnp.sqrt(jnp.sum(p * p, axis=1, keepdims=True))
    xn = xb / (xnorm + _EPS)
    pn = p / (pnorm + _EPS)
    sim_t = jnp.abs(jax.lax.dot_general(
        pn, xn, (((1,), (1,)), ((), ())),
        preferred_element_type=jnp.float32))

    iota_e = jax.lax.broadcasted_iota(jnp.int32, (e, tn), 0)
    m1 = jnp.max(sim_t, axis=0, keepdims=True)
    idx1 = jnp.min(jnp.where(sim_t == m1, iota_e, e), axis=0, keepdims=True)
    masked = jnp.where(iota_e == idx1, -1.0, sim_t)
    m2 = jnp.max(masked, axis=0, keepdims=True)
    idx2 = jnp.min(jnp.where(masked == m2, iota_e, e), axis=0, keepdims=True)
    c1 = jax.nn.sigmoid(m1 - m2)
    c2 = jax.nn.sigmoid(m2 - m1)
    w_t = (jnp.where(iota_e == idx1, c1, 0.0)
           + jnp.where(iota_e == idx2, c2, 0.0))

    row_e = jax.lax.broadcasted_iota(jnp.int32, (e, er), 0)
    col_e = jax.lax.broadcasted_iota(jnp.int32, (e, er), 1) // rank
    exp_m = (row_e == col_e).astype(jnp.float32)
    w = jax.lax.dot_general(
        w_t, exp_m, (((0,), (0,)), ((), ())),
        precision=jax.lax.Precision.HIGHEST,
        preferred_element_type=jnp.float32)

    z = jax.lax.dot_general(
        xb.astype(jnp.bfloat16), a_ref[:, :], (((1,), (1,)), ((), ())),
        preferred_element_type=jnp.float32)
    u = (z * w).astype(jnp.bfloat16)
    delta = jax.lax.dot_general(
        u, b_ref[:, :], (((1,), (0,)), ((), ())),
        preferred_element_type=jnp.float32)
    o_ref[:, :] = delta


def kernel(x, lora_A, lora_B, prototypes, scaling):
    bsz, seq, d = x.shape
    e, r, _ = lora_A.shape
    n = bsz * seq
    flat_x = x.reshape(n, d)
    a_stack = lora_A.reshape(e * r, d).astype(jnp.bfloat16)
    b_stack = (lora_B.transpose(0, 2, 1).reshape(e * r, d)
               * jnp.float32(scaling)).astype(jnp.bfloat16)

    tn = 1024
    grid = (n // tn,)
    out = pl.pallas_call(
        functools.partial(_fused_block, rank=r),
        grid=grid,
        in_specs=[
            pl.BlockSpec((tn, d), lambda i: (i, 0)),
            pl.BlockSpec((e, d), lambda i: (0, 0)),
            pl.BlockSpec((e * r, d), lambda i: (0, 0)),
            pl.BlockSpec((e * r, d), lambda i: (0, 0)),
        ],
        out_specs=pl.BlockSpec((tn, d), lambda i: (i, 0)),
        out_shape=jax.ShapeDtypeStruct((n, d), jnp.float32),
        compiler_params=pltpu.CompilerParams(
            dimension_semantics=("parallel",)),
    )(flat_x, prototypes, a_stack, b_stack)
    return out.reshape(bsz, seq, d)


# speedup vs baseline: 1.0824x; 1.0824x over previous
import functools

import jax
import jax.numpy as jnp
from jax.experimental import pallas as pl
from jax.experimental.pallas import tpu as pltpu

_EPS = 1e-8


def _fused_block(x_ref, p_ref, a_ref, b_ref, o_ref, *, rank):
    xb = x_ref[:, :]
    p = p_ref[:, :]
    tn = xb.shape[0]
    e = p.shape[0]
    er = a_ref.shape[0]

    xnorm = jnp.sqrt(jnp.sum(xb * xb, axis=1, keepdims=True))
    pnorm = jnp.sqrt(jnp.sum(p * p, axis=1, keepdims=True))
    xn = xb / (xnorm + _EPS)
    pn = p / (pnorm + _EPS)
    sim_t = jnp.abs(jax.lax.dot_general(
        pn, xn, (((1,), (1,)), ((), ())),
        preferred_element_type=jnp.float32))

    iota_e = jax.lax.broadcasted_iota(jnp.int32, (e, tn), 0)
    m1 = jnp.max(sim_t, axis=0, keepdims=True)
    idx1 = jnp.min(jnp.where(sim_t == m1, iota_e, e), axis=0, keepdims=True)
    masked = jnp.where(iota_e == idx1, -1.0, sim_t)
    m2 = jnp.max(masked, axis=0, keepdims=True)
    idx2 = jnp.min(jnp.where(masked == m2, iota_e, e), axis=0, keepdims=True)
    c1 = jax.nn.sigmoid(m1 - m2)
    c2 = jax.nn.sigmoid(m2 - m1)
    w_t = (jnp.where(iota_e == idx1, c1, 0.0)
           + jnp.where(iota_e == idx2, c2, 0.0))

    z_t = jax.lax.dot_general(
        a_ref[:, :], xb.astype(jnp.bfloat16), (((1,), (1,)), ((), ())),
        preferred_element_type=jnp.float32)
    u_t = jnp.concatenate(
        [z_t[ei * rank:(ei + 1) * rank, :] * w_t[ei:ei + 1, :]
         for ei in range(e)], axis=0).astype(jnp.bfloat16)
    delta = jax.lax.dot_general(
        u_t, b_ref[:, :], (((0,), (0,)), ((), ())),
        preferred_element_type=jnp.float32)
    o_ref[:, :] = delta


def kernel(x, lora_A, lora_B, prototypes, scaling):
    bsz, seq, d = x.shape
    e, r, _ = lora_A.shape
    n = bsz * seq
    flat_x = x.reshape(n, d)
    a_stack = lora_A.reshape(e * r, d).astype(jnp.bfloat16)
    b_stack = (lora_B.transpose(0, 2, 1).reshape(e * r, d)
               * jnp.float32(scaling)).astype(jnp.bfloat16)

    tn = 1024
    grid = (n // tn,)
    out = pl.pallas_call(
        functools.partial(_fused_block, rank=r),
        grid=grid,
        in_specs=[
            pl.BlockSpec((tn, d), lambda i: (i, 0)),
            pl.BlockSpec((e, d), lambda i: (0, 0)),
            pl.BlockSpec((e * r, d), lambda i: (0, 0)),
            pl.BlockSpec((e * r, d), lambda i: (0, 0)),
        ],
        out_specs=pl.BlockSpec((tn, d), lambda i: (i, 0)),
        out_shape=jax.ShapeDtypeStruct((n, d), jnp.float32),
        compiler_params=pltpu.CompilerParams(
            dimension_semantics=("parallel",)),
    )(flat_x, prototypes, a_stack, b_stack)
    return out.reshape(bsz, seq, d)


# speedup vs baseline: 1.0852x; 1.0025x over previous
import functools

import jax
import jax.numpy as jnp
from jax.experimental import pallas as pl
from jax.experimental.pallas import tpu as pltpu

_EPS = 1e-8


def _fused_block(x_ref, p_ref, a_ref, b_ref, o_ref, *, rank):
    xb = x_ref[:, :]
    p = p_ref[:, :]
    tn = xb.shape[0]
    e = p.shape[0]
    er = a_ref.shape[0]

    xnorm = jnp.sqrt(jnp.sum(xb * xb, axis=1, keepdims=True))
    pnorm = jnp.sqrt(jnp.sum(p * p, axis=1, keepdims=True))
    xn = (xb / (xnorm + _EPS)).astype(jnp.bfloat16)
    pn = (p / (pnorm + _EPS)).astype(jnp.bfloat16)
    sim_t = jnp.abs(jax.lax.dot_general(
        pn, xn, (((1,), (1,)), ((), ())),
        preferred_element_type=jnp.float32))

    iota_e = jax.lax.broadcasted_iota(jnp.int32, (e, tn), 0)
    m1 = jnp.max(sim_t, axis=0, keepdims=True)
    idx1 = jnp.min(jnp.where(sim_t == m1, iota_e, e), axis=0, keepdims=True)
    masked = jnp.where(iota_e == idx1, -1.0, sim_t)
    m2 = jnp.max(masked, axis=0, keepdims=True)
    idx2 = jnp.min(jnp.where(masked == m2, iota_e, e), axis=0, keepdims=True)
    c1 = jax.nn.sigmoid(m1 - m2)
    c2 = jax.nn.sigmoid(m2 - m1)
    w_t = (jnp.where(iota_e == idx1, c1, 0.0)
           + jnp.where(iota_e == idx2, c2, 0.0))

    z_t = jax.lax.dot_general(
        a_ref[:, :], xb.astype(jnp.bfloat16), (((1,), (1,)), ((), ())),
        preferred_element_type=jnp.float32)
    u_t = jnp.concatenate(
        [z_t[ei * rank:(ei + 1) * rank, :] * w_t[ei:ei + 1, :]
         for ei in range(e)], axis=0).astype(jnp.bfloat16)
    delta = jax.lax.dot_general(
        u_t, b_ref[:, :], (((0,), (0,)), ((), ())),
        preferred_element_type=jnp.float32)
    o_ref[:, :] = delta


def kernel(x, lora_A, lora_B, prototypes, scaling):
    bsz, seq, d = x.shape
    e, r, _ = lora_A.shape
    n = bsz * seq
    flat_x = x.reshape(n, d)
    a_stack = lora_A.reshape(e * r, d).astype(jnp.bfloat16)
    b_stack = (lora_B.transpose(0, 2, 1).reshape(e * r, d)
               * jnp.float32(scaling)).astype(jnp.bfloat16)

    tn = 1024
    grid = (n // tn,)
    out = pl.pallas_call(
        functools.partial(_fused_block, rank=r),
        grid=grid,
        in_specs=[
            pl.BlockSpec((tn, d), lambda i: (i, 0)),
            pl.BlockSpec((e, d), lambda i: (0, 0)),
            pl.BlockSpec((e * r, d), lambda i: (0, 0)),
            pl.BlockSpec((e * r, d), lambda i: (0, 0)),
        ],
        out_specs=pl.BlockSpec((tn, d), lambda i: (i, 0)),
        out_shape=jax.ShapeDtypeStruct((n, d), jnp.float32),
        compiler_params=pltpu.CompilerParams(
            dimension_semantics=("parallel",)),
    )(flat_x, prototypes, a_stack, b_stack)
    return out.reshape(bsz, seq, d)
